# Initial kernel scaffold; baseline (speedup 1.0000x reference)
import jax
import jax.numpy as jnp
from jax.experimental import pallas as pl


def kernel(atom_features, f_bonds, a2a, a2b, graph_ids, params):
    raise NotImplementedError("write your pallas kernel here")


# speedup vs baseline: 2.7486x; 2.7486x over previous
import functools

import jax
import jax.numpy as jnp
from jax import lax
from jax.experimental import pallas as pl
from jax.experimental.pallas import tpu as pltpu
from jax.experimental.pallas import tpu_sc as plsc

_WR = 1e-6
_DR = 1e-5

_NC = 2
_NS = 16
_NW = _NC * _NS
_CH = 128


def _sc_gather(table, idx, D):
    E = idx.shape[0]
    nchunk = E // _CH
    per_w = (nchunk + _NW - 1) // _NW
    mesh = plsc.VectorSubcoreMesh(core_axis_name="c", subcore_axis_name="s")

    @functools.partial(
        pl.kernel,
        out_type=jax.ShapeDtypeStruct((E, D), jnp.float32),
        mesh=mesh,
        scratch_types=[
            pltpu.VMEM((_CH,), jnp.int32),
            pltpu.VMEM((_CH, D), jnp.float32),
            pltpu.SemaphoreType.DMA,
        ],
        compiler_params=pltpu.CompilerParams(use_tc_tiling_on_sc=(D % 128 == 0)),
    )
    def gk(table_hbm, idx_hbm, out_hbm, idx_v, rows_v, sem):
        wid = lax.axis_index("s") * _NC + lax.axis_index("c")

        def body(i, carry):
            c = wid + i * _NW

            @pl.when(c < nchunk)
            def _():
                base = c * _CH
                pltpu.sync_copy(idx_hbm.at[pl.ds(base, _CH)], idx_v)
                pltpu.async_copy(table_hbm.at[idx_v], rows_v, sem).wait()
                pltpu.sync_copy(rows_v, out_hbm.at[pl.ds(base, _CH)])

            return carry

        lax.fori_loop(0, per_w, body, 0)

    return gk(table, idx)


def _proj_body(x_ref, w_ref, b_ref, o_ref):
    o_ref[...] = (
        jnp.dot(x_ref[...], w_ref[...], preferred_element_type=jnp.float32)
        + b_ref[...]
    )


def _proj(x, w, b):
    n, _ = x.shape
    h = w.shape[1]
    bn = 2000
    return pl.pallas_call(
        _proj_body,
        grid=(n // bn,),
        in_specs=[
            pl.BlockSpec((bn, x.shape[1]), lambda i: (i, 0)),
            pl.BlockSpec(w.shape, lambda i: (0, 0)),
            pl.BlockSpec((1, h), lambda i: (0, 0)),
        ],
        out_specs=pl.BlockSpec((bn, h), lambda i: (i, 0)),
        out_shape=jax.ShapeDtypeStruct((n, h), jnp.float32),
    )(x, w, b.reshape(1, h))


def _combine_body(
    nfa_ref, nfb_ref, fa_ref,
    wa0_ref, wa1_ref, w01_ref, w02_ref, wleiA_ref, wnl1_ref, wnl2_ref,
    wb0_ref, wb1_ref, wleiB_ref, bias_ref,
    fnew_ref, ah_ref, *, bn, kk, h,
):
    dot = lambda x, w: jax.lax.dot_general(
        x, w, (((1,), (0,)), ((), ())), preferred_element_type=jnp.float32
    )
    nfa = nfa_ref[...]
    nfb = nfb_ref[...]
    fa = fa_ref[...]
    b = bias_ref[...]

    h_ab = (dot(nfa, wa0_ref[...]) + b[0:1, :]) * (dot(nfb, wb0_ref[...]) + b[1:2, :])
    gpre = dot(nfa, wa1_ref[...]) + dot(nfb, wb1_ref[...]) + b[2:3, :]
    gs = dot(fa, w01_ref[...])
    gpre3 = gpre.reshape(bn, kk, h) + gs[:, None, :]
    g = jax.nn.sigmoid(gpre3) * 10.0
    f_nei = jnp.sum(g * h_ab.reshape(bn, kk, h), axis=1)
    f_self = dot(fa, w02_ref[...]) + b[3:4, :]
    ah_ref[...] = f_nei * f_self

    lei = jnp.maximum(dot(nfa, wleiA_ref[...]) + dot(nfb, wleiB_ref[...]) + b[4:5, :], 0.0)
    nl = jnp.sum(lei.reshape(bn, kk, h), axis=1)
    fnew_ref[...] = jnp.maximum(dot(fa, wnl1_ref[...]) + dot(nl, wnl2_ref[...]) + b[5:6, :], 0.0)


def _combine(nfa, nfb, fa, ws, bias_stack):
    n, h = fa.shape
    e, bf = nfb.shape
    kk = e // n
    bn = 400
    be = bn * kk
    grid = (n // bn,)
    full = lambda a: pl.BlockSpec(a.shape, lambda i: (0, 0))
    kernel_fn = functools.partial(_combine_body, bn=bn, kk=kk, h=h)
    return pl.pallas_call(
        kernel_fn,
        grid=grid,
        in_specs=[
            pl.BlockSpec((be, h), lambda i: (i, 0)),
            pl.BlockSpec((be, bf), lambda i: (i, 0)),
            pl.BlockSpec((bn, h), lambda i: (i, 0)),
            *[full(w) for w in ws],
            pl.BlockSpec((8, h), lambda i: (0, 0)),
        ],
        out_specs=[
            pl.BlockSpec((bn, h), lambda i: (i, 0)),
            pl.BlockSpec((bn, h), lambda i: (i, 0)),
        ],
        out_shape=[
            jax.ShapeDtypeStruct((n, h), jnp.float32),
            jax.ShapeDtypeStruct((n, h), jnp.float32),
        ],
    )(nfa, nfb, fa, *ws, bias_stack)


def _readout_body(
    ah_ref, gid_ref, u1_ref, u2_ref,
    wo0_ref, bo0_ref, wo1_ref, bo1_ref, wmu_ref, bmu_ref, wlv_ref, blv_ref,
    plog_ref,
    mean_ref, lv_ref, reg_ref, *, nb, nseg,
):
    ah = ah_ref[...]
    gid = gid_ref[...]
    seg = lax.broadcasted_iota(jnp.int32, (nseg, gid.shape[1]), 0)
    oh = (seg == gid).astype(jnp.float32)
    sums = jax.lax.dot_general(
        oh, ah, (((1,), (0,)), ((), ())), preferred_element_type=jnp.float32
    )
    counts = jnp.sum(oh, axis=1, keepdims=True)
    mol = sums / jnp.maximum(counts, 1.0)

    eps = 1e-7
    temp = 0.1

    def drop(x, p, unif):
        dp = (
            jnp.log(p + eps) - jnp.log(1.0 - p + eps)
            + jnp.log(unif + eps) - jnp.log(1.0 - unif + eps)
        )
        mask = jax.nn.sigmoid(dp / temp)
        return x * (1.0 - mask) / (1.0 - p)

    p1 = jax.nn.sigmoid(plog_ref[0, 0])
    p2 = jax.nn.sigmoid(plog_ref[1, 0])
    x = drop(mol, p1, u1_ref[...])
    x = jnp.maximum(jnp.dot(x, wo0_ref[...], preferred_element_type=jnp.float32) + bo0_ref[...], 0.0)
    x = drop(x, p2, u2_ref[...])
    x = jnp.maximum(jnp.dot(x, wo1_ref[...], preferred_element_type=jnp.float32) + bo1_ref[...], 0.0)
    mean_ref[...] = jnp.dot(x, wmu_ref[...], preferred_element_type=jnp.float32) + bmu_ref[...]
    lv_ref[...] = jnp.dot(x, wlv_ref[...], preferred_element_type=jnp.float32) + blv_ref[...]

    d_in = jnp.float32(wo0_ref.shape[0])
    reg1 = (
        _WR * (jnp.sum(wo0_ref[...] ** 2) + jnp.sum(bo0_ref[...] ** 2)) / (1.0 - p1)
        + _DR * d_in * (p1 * jnp.log(p1) + (1.0 - p1) * jnp.log(1.0 - p1))
    )
    d_in2 = jnp.float32(wo1_ref.shape[0])
    reg2 = (
        _WR * (jnp.sum(wo1_ref[...] ** 2) + jnp.sum(bo1_ref[...] ** 2)) / (1.0 - p2)
        + _DR * d_in2 * (p2 * jnp.log(p2) + (1.0 - p2) * jnp.log(1.0 - p2))
    )
    reg_ref[...] = jnp.full((1, 1), reg1 + reg2, jnp.float32)


def _readout(ah, gid_row, u1, u2, p):
    n, h = ah.shape
    nseg = u1.shape[0]
    r = p["Wo0"].shape[1]
    plog = jnp.stack([p["plog1"], p["plog2"]]).reshape(2, 1)
    args = [
        ah, gid_row, u1, u2,
        p["Wo0"], p["bo0"].reshape(1, r), p["Wo1"], p["bo1"].reshape(1, r),
        p["Wmu"], p["bmu"].reshape(1, 1), p["Wlv"], p["blv"].reshape(1, 1),
        plog,
    ]
    full = lambda a: pl.BlockSpec(a.shape, lambda: (0, 0))
    kernel_fn = functools.partial(_readout_body, nb=n, nseg=nseg)
    return pl.pallas_call(
        kernel_fn,
        in_specs=[full(a) for a in args],
        out_specs=[
            pl.BlockSpec((nseg, 1), lambda: (0, 0)),
            pl.BlockSpec((nseg, 1), lambda: (0, 0)),
            pl.BlockSpec((1, 1), lambda: (0, 0)),
        ],
        out_shape=[
            jax.ShapeDtypeStruct((nseg, 1), jnp.float32),
            jax.ShapeDtypeStruct((nseg, 1), jnp.float32),
            jax.ShapeDtypeStruct((1, 1), jnp.float32),
        ],
    )(*args)


def kernel(atom_features, f_bonds, a2a, a2b, graph_ids, params):
    p = params
    n, _ = atom_features.shape
    e, bf = f_bonds.shape
    kk = a2a.shape[1]
    h = p["W00"].shape[1]
    depth = 3
    nseg = 64

    a2a_f = a2a.reshape(-1).astype(jnp.int32)
    a2b_f = a2b.reshape(-1).astype(jnp.int32)

    bias_stack = jnp.zeros((8, h), jnp.float32)
    bias_stack = bias_stack.at[0].set(p["ba0"])
    bias_stack = bias_stack.at[1].set(p["bb0"])
    bias_stack = bias_stack.at[2].set(p["ba1"] + p["bb1"] + p["b01"])
    bias_stack = bias_stack.at[3].set(p["b02"])
    bias_stack = bias_stack.at[4].set(p["blei"])
    bias_stack = bias_stack.at[5].set(p["bnl"])
    ws = [
        p["Wa0"], p["Wa1"], p["W01"], p["W02"],
        p["Wlei"][:h, :], p["Wnl"][:h, :], p["Wnl"][h:, :],
        p["Wb0"], p["Wb1"], p["Wlei"][h:, :],
    ]

    f_atoms = _proj(atom_features, p["W00"], p["b00"])
    nfb = _sc_gather(f_bonds, a2b_f, bf)
    ah = f_atoms
    for _ in range(depth):
        nfa = _sc_gather(f_atoms, a2a_f, h)
        f_atoms, ah = _combine(nfa, nfb, f_atoms, ws, bias_stack)

    kd = jax.random.key(42)
    u1 = jax.random.uniform(jax.random.fold_in(kd, 1), (nseg, 1), dtype=jnp.float32)
    u2 = jax.random.uniform(jax.random.fold_in(kd, 2), (nseg, 1), dtype=jnp.float32)
    gid_row = graph_ids.reshape(1, n).astype(jnp.int32)

    mean, log_var, reg = _readout(ah, gid_row, u1, u2, p)
    return mean, log_var, reg[0, 0]
